# baseline (speedup 1.0000x reference)
import functools

import jax
import jax.numpy as jnp
from jax import lax
from jax.experimental import pallas as pl
from jax.experimental.pallas import tpu as pltpu
from jax.experimental.pallas import tpu_sc as plsc

_DEPTH_SCALES = 4

_CHUNK = 8


def _sc_gather_call(table1d, idxg, n_rows, row_words):
    info = plsc.get_sparse_core_info()
    nw = info.num_cores * info.num_subcores
    bpw = n_rows // nw
    n_chunks = bpw // _CHUNK
    n_pairs = n_chunks // 2
    mesh = plsc.VectorSubcoreMesh(core_axis_name="c", subcore_axis_name="s")

    @functools.partial(
        pl.kernel,
        out_type=jax.ShapeDtypeStruct((n_rows * row_words,), jnp.float32),
        mesh=mesh,
        scratch_types=[
            pltpu.SMEM((bpw,), jnp.int32),
            pltpu.VMEM_SHARED((16, bpw), jnp.int32),
            pltpu.VMEM((_CHUNK * row_words,), jnp.float32),
            pltpu.VMEM((_CHUNK * row_words,), jnp.float32),
            pltpu.SemaphoreType.DMA((2, _CHUNK)),
            pltpu.SemaphoreType.DMA((2,)),
        ],
    )
    def k(table_hbm, idx_hbm, out_hbm, idx_s, idx_v, buf0, buf1, gsem, ssem):
        sid = lax.axis_index("s")
        wid = sid * info.num_cores + lax.axis_index("c")
        base = wid * bpw
        pltpu.sync_copy(idx_hbm.at[wid], idx_v.at[sid])
        pltpu.sync_copy(idx_v.at[sid], idx_s)
        bufs = (buf0, buf1)

        def gather(ch, p, j):
            row = ch * _CHUNK + j
            src = table_hbm.at[pl.ds(idx_s[row] * row_words, row_words)]
            dst = bufs[p].at[pl.ds(j * row_words, row_words)]
            return pltpu.make_async_copy(src, dst, gsem.at[p, j])

        def scatter(ch, p):
            dst = out_hbm.at[pl.ds((base + ch * _CHUNK) * row_words,
                                   _CHUNK * row_words)]
            return pltpu.make_async_copy(bufs[p], dst, ssem.at[p])

        for j in range(_CHUNK):
            gather(0, 0, j).start()

        @pl.loop(0, n_pairs)
        def _(m):
            ch0 = 2 * m
            ch1 = 2 * m + 1

            @pl.when(m > 0)
            def _():
                scatter(ch1 - 2, 1).wait()

            for j in range(_CHUNK):
                gather(ch1, 1, j).start()

            for j in range(_CHUNK):
                gather(ch0, 0, j).wait()
            scatter(ch0, 0).start()

            @pl.when(m < n_pairs - 1)
            def _():
                scatter(ch0, 0).wait()
                for j in range(_CHUNK):
                    gather(ch0 + 2, 0, j).start()

            for j in range(_CHUNK):
                gather(ch1, 1, j).wait()
            scatter(ch1, 1).start()

        scatter(n_chunks - 2, 0).wait()
        scatter(n_chunks - 1, 1).wait()

    return k(table1d, idxg.reshape(nw, bpw))


def _pool_body(x_ref, avg_ref, mx_ref):
    j = pl.program_id(1)
    xb = x_ref[0]
    cb = xb.shape[0]
    avg_ref[0, 0, pl.ds(j * cb, cb)] = jnp.sum(xb, axis=1) / xb.shape[1]
    mx_ref[0, 0, pl.ds(j * cb, cb)] = jnp.max(xb, axis=1)


def _pool_call(x2, n, c, hw, cb=256):
    avg, mx = pl.pallas_call(
        _pool_body,
        grid=(n, c // cb),
        in_specs=[pl.BlockSpec((1, cb, hw), lambda i, j: (i, j, 0))],
        out_specs=[pl.BlockSpec((1, 1, c), lambda i, j: (i, 0, 0)),
                   pl.BlockSpec((1, 1, c), lambda i, j: (i, 0, 0))],
        out_shape=[jax.ShapeDtypeStruct((n, 1, c), jnp.float32),
                   jax.ShapeDtypeStruct((n, 1, c), jnp.float32)],
    )(x2)
    return avg.reshape(n, c), mx.reshape(n, c)


def _rank_body(attn_ref, idx_ref):
    nb, dc = attn_ref.shape
    c = 768
    d = dc // c
    k = c // d
    ib = 128
    for s in range(d):
        v = attn_ref[:, pl.ds(s * c, c)]
        for i0 in range(0, c, ib):
            vi = attn_ref[:, pl.ds(s * c + i0, ib)]
            gt = (v[:, :, None] > vi[:, None, :]).astype(jnp.int32)
            jlt = jax.lax.broadcasted_iota(jnp.int32, (1, c, 1), 1) < (
                i0 + jax.lax.broadcasted_iota(jnp.int32, (1, 1, ib), 2))
            eq = ((v[:, :, None] == vi[:, None, :]) & jlt).astype(jnp.int32)
            rank = jnp.sum(gt + eq, axis=1)
            pidx = jax.lax.broadcasted_iota(jnp.int32, (1, 1, k), 2)
            hit = (rank[:, :, None] == pidx).astype(jnp.int32)
            contrib = jnp.sum(
                hit * (i0 + jax.lax.broadcasted_iota(jnp.int32, (1, ib, 1), 1)),
                axis=1)
            if i0 == 0:
                idx_ref[:, pl.ds(s * k, k)] = contrib
            else:
                idx_ref[:, pl.ds(s * k, k)] += contrib
    row = jax.lax.broadcasted_iota(jnp.int32, (nb, c), 0)
    idx_ref[...] += row * c


def _rank_call(attn_t, n, c):
    dc = attn_t.shape[1]
    return pl.pallas_call(
        _rank_body,
        in_specs=[pl.BlockSpec((n, dc), lambda: (0, 0))],
        out_specs=pl.BlockSpec((n, c), lambda: (0, 0)),
        out_shape=jax.ShapeDtypeStruct((n, c), jnp.int32),
    )(attn_t)


def kernel(x, W1, b1, W2, b2):
    n, c, h, w = x.shape
    d = _DEPTH_SCALES
    x2 = x.reshape(n, c, h * w)
    avg, mx = _pool_call(x2, n, c, h * w)

    def mlp(v):
        hdn = jnp.maximum(v @ W1 + b1, 0.0)
        return hdn @ W2 + b2

    attn = jax.nn.sigmoid(mlp(avg) + mlp(mx))
    attn_t = jnp.transpose(attn.reshape(n, c, d), (0, 2, 1)).reshape(n, d * c)
    idxg = _rank_call(attn_t, n, c).reshape(-1)
    out = _sc_gather_call(x.reshape(-1), idxg, n * c, h * w)
    return out.reshape(n, c, h, w)
